# Initial kernel scaffold; baseline (speedup 1.0000x reference)
import jax
import jax.numpy as jnp
from jax.experimental import pallas as pl


def kernel(_, decoder_input_ids, probs):
    raise NotImplementedError("write your pallas kernel here")


# speedup vs baseline: 2.8877x; 2.8877x over previous
import functools

import jax
import jax.numpy as jnp
from jax import lax
from jax.experimental import pallas as pl
from jax.experimental.pallas import tpu as pltpu
from jax.experimental.pallas import tpu_sc as plsc

DIM = 128
B = 1024
LOUT = 199
N = B * LOUT
NW = 32
PER_W = N // NW
GROUP = 128
NG = PER_W // GROUP + 1
LAST_OFF = PER_W - GROUP


def _make_gather():
    mesh = plsc.VectorSubcoreMesh(core_axis_name="c", subcore_axis_name="s")

    @functools.partial(
        pl.kernel,
        mesh=mesh,
        out_type=jax.ShapeDtypeStruct((N, DIM), jnp.float32),
        scratch_types=[
            pltpu.VMEM((NG, GROUP), jnp.int32),
            pltpu.VMEM((GROUP, DIM), jnp.float32),
            pltpu.SemaphoreType.DMA,
        ],
    )
    def gather_kernel(idx_hbm, table_hbm, out_hbm, idx_v, rows_v, sem):
        wid = lax.axis_index("s") * 2 + lax.axis_index("c")
        base = wid * PER_W
        pltpu.sync_copy(idx_hbm.at[wid], idx_v)

        def body(g, carry):
            row_off = jnp.where(g == NG - 1, LAST_OFF, g * GROUP)
            pltpu.async_copy(table_hbm.at[idx_v.at[g]], rows_v, sem).wait()
            pltpu.sync_copy(rows_v, out_hbm.at[pl.ds(base + row_off, GROUP)])
            return carry

        lax.fori_loop(0, NG, body, 0)

    return gather_kernel


_gather = _make_gather()


def kernel(_, decoder_input_ids, probs):
    ids = decoder_input_ids[:, 1:].reshape(NW, PER_W)
    groups = jnp.concatenate(
        [
            ids[:, : (NG - 1) * GROUP].reshape(NW, NG - 1, GROUP),
            ids[:, LAST_OFF:].reshape(NW, 1, GROUP),
        ],
        axis=1,
    )
    out = _gather(groups, probs)
    return (out.reshape(B, LOUT, DIM),)


# speedup vs baseline: 3.2941x; 1.1407x over previous
import functools

import jax
import jax.numpy as jnp
from jax import lax
from jax.experimental import pallas as pl
from jax.experimental.pallas import tpu as pltpu
from jax.experimental.pallas import tpu_sc as plsc

DIM = 128
B = 1024
LOUT = 199
N = B * LOUT
NW = 32
PER_W = N // NW
GROUP = 128
NG = PER_W // GROUP + 1
LAST_OFF = PER_W - GROUP


NBUF = 5
NROUND = NG // NBUF


def _make_gather():
    mesh = plsc.VectorSubcoreMesh(core_axis_name="c", subcore_axis_name="s")

    @functools.partial(
        pl.kernel,
        mesh=mesh,
        out_type=jax.ShapeDtypeStruct((N, DIM), jnp.float32),
        scratch_types=[
            pltpu.VMEM((NG, GROUP), jnp.int32),
        ]
        + [pltpu.VMEM((GROUP, DIM), jnp.float32) for _ in range(NBUF)]
        + [pltpu.SemaphoreType.DMA for _ in range(2 * NBUF)],
    )
    def gather_kernel(idx_hbm, table_hbm, out_hbm, idx_v, *bufs_and_sems):
        bufs = bufs_and_sems[:NBUF]
        gsem = bufs_and_sems[NBUF : 2 * NBUF]
        ssem = bufs_and_sems[2 * NBUF :]
        wid = lax.axis_index("s") * 2 + lax.axis_index("c")
        base = wid * PER_W
        pltpu.sync_copy(idx_hbm.at[wid], idx_v)

        def out_slab(g):
            row_off = jnp.where(g == NG - 1, LAST_OFF, g * GROUP)
            return out_hbm.at[pl.ds(base + row_off, GROUP)]

        def body(t, carry):
            for b in range(NBUF):
                g = t * NBUF + b

                @pl.when(t > 0)
                def _():
                    pltpu.make_async_copy(bufs[b], out_slab(g - NBUF), ssem[b]).wait()

                pltpu.async_copy(table_hbm.at[idx_v.at[g]], bufs[b], gsem[b])
            for b in range(NBUF):
                g = t * NBUF + b
                pltpu.make_async_copy(
                    table_hbm.at[idx_v.at[g]], bufs[b], gsem[b]
                ).wait()
                pltpu.async_copy(bufs[b], out_slab(g), ssem[b])
            return carry

        lax.fori_loop(0, NROUND, body, 0)
        for b in range(NBUF):
            g = (NROUND - 1) * NBUF + b
            pltpu.make_async_copy(bufs[b], out_slab(g), ssem[b]).wait()

    return gather_kernel


_gather = _make_gather()


def kernel(_, decoder_input_ids, probs):
    ids = decoder_input_ids[:, 1:].reshape(NW, PER_W)
    groups = jnp.concatenate(
        [
            ids[:, : (NG - 1) * GROUP].reshape(NW, NG - 1, GROUP),
            ids[:, LAST_OFF:].reshape(NW, 1, GROUP),
        ],
        axis=1,
    )
    out = _gather(groups, probs)
    return (out.reshape(B, LOUT, DIM),)


# speedup vs baseline: 4.6091x; 1.3992x over previous
import functools

import jax
import jax.numpy as jnp
from jax import lax
from jax.experimental import pallas as pl
from jax.experimental.pallas import tpu as pltpu
from jax.experimental.pallas import tpu_sc as plsc

DIM = 128
B = 1024
LOUT = 199
NW = 32
BPW = B // NW
G0 = 128
OFF1 = LOUT - G0
NBUF = 4
NROUND = BPW // NBUF


def _make_gather():
    mesh = plsc.VectorSubcoreMesh(core_axis_name="c", subcore_axis_name="s")

    @functools.partial(
        pl.kernel,
        mesh=mesh,
        out_type=jax.ShapeDtypeStruct((B, LOUT, DIM), jnp.float32),
        scratch_types=[
            pltpu.VMEM((BPW, 2, G0), jnp.int32),
        ]
        + [pltpu.VMEM((LOUT, DIM), jnp.float32) for _ in range(NBUF)]
        + [pltpu.SemaphoreType.DMA for _ in range(2 * NBUF)],
    )
    def gather_kernel(idx_hbm, table_hbm, out_hbm, idx_v, *bufs_and_sems):
        bufs = bufs_and_sems[:NBUF]
        gsem = bufs_and_sems[NBUF : 2 * NBUF]
        ssem = bufs_and_sems[2 * NBUF :]
        wid = lax.axis_index("s") * 2 + lax.axis_index("c")
        base = wid * BPW
        pltpu.sync_copy(idx_hbm.at[wid], idx_v)

        def body(t, carry):
            for b in range(NBUF):
                i = t * NBUF + b

                @pl.when(t > 0)
                def _():
                    pltpu.make_async_copy(bufs[b], out_hbm.at[base], ssem[b]).wait()

                pltpu.async_copy(
                    table_hbm.at[idx_v.at[i, 0]], bufs[b].at[pl.ds(0, G0)], gsem[b]
                )
                pltpu.async_copy(
                    table_hbm.at[idx_v.at[i, 1]],
                    bufs[b].at[pl.ds(OFF1, G0)],
                    gsem[b],
                )
            for b in range(NBUF):
                i = t * NBUF + b
                pltpu.make_async_copy(
                    table_hbm.at[idx_v.at[i, 0]], bufs[b].at[pl.ds(0, G0)], gsem[b]
                ).wait()
                pltpu.make_async_copy(
                    table_hbm.at[idx_v.at[i, 1]],
                    bufs[b].at[pl.ds(OFF1, G0)],
                    gsem[b],
                ).wait()
                pltpu.async_copy(bufs[b], out_hbm.at[base + i], ssem[b])
            return carry

        lax.fori_loop(0, NROUND, body, 0)
        for b in range(NBUF):
            pltpu.make_async_copy(bufs[b], out_hbm.at[base], ssem[b]).wait()

    return gather_kernel


_gather = _make_gather()


def kernel(_, decoder_input_ids, probs):
    ids = decoder_input_ids[:, 1:]
    g0 = ids[:, :G0]
    g1 = ids[:, OFF1:]
    groups = jnp.stack([g0, g1], axis=1).reshape(NW, BPW, 2, G0)
    out = _gather(groups, probs)
    return (out,)


# speedup vs baseline: 4.6201x; 1.0024x over previous
import functools

import jax
import jax.numpy as jnp
from jax import lax
from jax.experimental import pallas as pl
from jax.experimental.pallas import tpu as pltpu
from jax.experimental.pallas import tpu_sc as plsc

DIM = 128
B = 1024
LOUT = 199
NW = 32
BPW = B // NW
G0 = 128
OFF1 = LOUT - G0
NBUF = 4
NROUND = BPW // NBUF


def _make_gather():
    mesh = plsc.VectorSubcoreMesh(core_axis_name="c", subcore_axis_name="s")

    @functools.partial(
        pl.kernel,
        mesh=mesh,
        out_type=jax.ShapeDtypeStruct((B, LOUT, DIM), jnp.float32),
        compiler_params=pltpu.CompilerParams(use_tc_tiling_on_sc=True),
        scratch_types=[
            pltpu.VMEM((BPW, 2, G0), jnp.int32),
        ]
        + [pltpu.VMEM((LOUT, DIM), jnp.float32) for _ in range(NBUF)]
        + [pltpu.SemaphoreType.DMA for _ in range(2 * NBUF)],
    )
    def gather_kernel(idx_hbm, table_hbm, out_hbm, idx_v, *bufs_and_sems):
        bufs = bufs_and_sems[:NBUF]
        gsem = bufs_and_sems[NBUF : 2 * NBUF]
        ssem = bufs_and_sems[2 * NBUF :]
        wid = lax.axis_index("s") * 2 + lax.axis_index("c")
        base = wid * BPW
        pltpu.sync_copy(idx_hbm.at[wid], idx_v)

        def body(t, carry):
            for b in range(NBUF):
                i = t * NBUF + b

                @pl.when(t > 0)
                def _():
                    pltpu.make_async_copy(bufs[b], out_hbm.at[base], ssem[b]).wait()

                pltpu.async_copy(
                    table_hbm.at[idx_v.at[i, 0]], bufs[b].at[pl.ds(0, G0)], gsem[b]
                )
                pltpu.async_copy(
                    table_hbm.at[idx_v.at[i, 1]],
                    bufs[b].at[pl.ds(OFF1, G0)],
                    gsem[b],
                )
            for b in range(NBUF):
                i = t * NBUF + b
                pltpu.make_async_copy(
                    table_hbm.at[idx_v.at[i, 0]], bufs[b].at[pl.ds(0, G0)], gsem[b]
                ).wait()
                pltpu.make_async_copy(
                    table_hbm.at[idx_v.at[i, 1]],
                    bufs[b].at[pl.ds(OFF1, G0)],
                    gsem[b],
                ).wait()
                pltpu.async_copy(bufs[b], out_hbm.at[base + i], ssem[b])
            return carry

        lax.fori_loop(0, NROUND, body, 0)
        for b in range(NBUF):
            pltpu.make_async_copy(bufs[b], out_hbm.at[base], ssem[b]).wait()

    return gather_kernel


_gather = _make_gather()


def kernel(_, decoder_input_ids, probs):
    ids = decoder_input_ids[:, 1:]
    g0 = ids[:, :G0]
    g1 = ids[:, OFF1:]
    groups = jnp.stack([g0, g1], axis=1).reshape(NW, BPW, 2, G0)
    out = _gather(groups, probs)
    return (out,)


# speedup vs baseline: 8.5365x; 1.8477x over previous
import functools

import jax
import jax.numpy as jnp
from jax import lax
from jax.experimental import pallas as pl
from jax.experimental.pallas import tpu as pltpu
from jax.experimental.pallas import tpu_sc as plsc

DIM = 128
B = 1024
LOUT = 199
NW = 32
G = 128
CPL = B // G
NITEM = LOUT * CPL
IPW = 50
NBUF = 5
NROUND = IPW // NBUF


def _make_gather():
    mesh = plsc.VectorSubcoreMesh(core_axis_name="c", subcore_axis_name="s")

    @functools.partial(
        pl.kernel,
        mesh=mesh,
        out_type=jax.ShapeDtypeStruct((LOUT, B, DIM), jnp.float32),
        compiler_params=pltpu.CompilerParams(use_tc_tiling_on_sc=True),
        scratch_types=[
            pltpu.VMEM((IPW, G), jnp.int32),
        ]
        + [pltpu.VMEM((G, DIM), jnp.float32) for _ in range(NBUF)]
        + [pltpu.SemaphoreType.DMA for _ in range(2 * NBUF)],
    )
    def gather_kernel(idx_hbm, table_hbm, out_hbm, idx_v, *bufs_and_sems):
        bufs = bufs_and_sems[:NBUF]
        gsem = bufs_and_sems[NBUF : 2 * NBUF]
        ssem = bufs_and_sems[2 * NBUF :]
        wid = lax.axis_index("s") * 2 + lax.axis_index("c")
        pltpu.sync_copy(idx_hbm.at[wid], idx_v)

        def out_slab(j):
            g = wid * IPW + j
            item = jnp.where(g < NITEM, g, g - NITEM)
            return out_hbm.at[item >> 3, pl.ds((item & 7) * G, G)]

        def body(t, carry):
            for b in range(NBUF):
                j = t * NBUF + b

                @pl.when(t > 0)
                def _():
                    pltpu.make_async_copy(bufs[b], out_slab(j - NBUF), ssem[b]).wait()

                pltpu.async_copy(table_hbm.at[idx_v.at[j]], bufs[b], gsem[b])
            for b in range(NBUF):
                j = t * NBUF + b
                pltpu.make_async_copy(
                    table_hbm.at[idx_v.at[j]], bufs[b], gsem[b]
                ).wait()
                pltpu.async_copy(bufs[b], out_slab(j), ssem[b])
            return carry

        lax.fori_loop(0, NROUND, body, 0)
        for b in range(NBUF):
            pltpu.make_async_copy(bufs[b], out_slab((NROUND - 1) * NBUF + b), ssem[b]).wait()

    return gather_kernel


_gather = _make_gather()


def kernel(_, decoder_input_ids, probs):
    ids_t = decoder_input_ids[:, 1:].T.reshape(NITEM, G)
    idx = jnp.concatenate([ids_t, ids_t[: NW * IPW - NITEM]]).reshape(NW, IPW, G)
    out_t = _gather(idx, probs)
    return (out_t.transpose(1, 0, 2),)


# speedup vs baseline: 8.6267x; 1.0106x over previous
import functools

import jax
import jax.numpy as jnp
from jax import lax
from jax.experimental import pallas as pl
from jax.experimental.pallas import tpu as pltpu
from jax.experimental.pallas import tpu_sc as plsc

DIM = 128
B = 1024
LOUT = 199
NW = 32
G = 64
CPL = B // G
NITEM = LOUT * CPL
IPW = 100
NBUF = 10
NROUND = IPW // NBUF


def _make_gather():
    mesh = plsc.VectorSubcoreMesh(core_axis_name="c", subcore_axis_name="s")

    @functools.partial(
        pl.kernel,
        mesh=mesh,
        out_type=jax.ShapeDtypeStruct((LOUT, B, DIM), jnp.float32),
        compiler_params=pltpu.CompilerParams(use_tc_tiling_on_sc=True),
        scratch_types=[
            pltpu.VMEM((IPW, G), jnp.int32),
        ]
        + [pltpu.VMEM((G, DIM), jnp.float32) for _ in range(NBUF)]
        + [pltpu.SemaphoreType.DMA for _ in range(2 * NBUF)],
    )
    def gather_kernel(idx_hbm, table_hbm, out_hbm, idx_v, *bufs_and_sems):
        bufs = bufs_and_sems[:NBUF]
        gsem = bufs_and_sems[NBUF : 2 * NBUF]
        ssem = bufs_and_sems[2 * NBUF :]
        wid = lax.axis_index("s") * 2 + lax.axis_index("c")
        pltpu.sync_copy(idx_hbm.at[wid], idx_v)

        def out_slab(j):
            g = wid * IPW + j
            item = jnp.where(g < NITEM, g, g - NITEM)
            return out_hbm.at[item >> 4, pl.ds((item & 15) * G, G)]

        def body(t, carry):
            for b in range(NBUF):
                j = t * NBUF + b

                @pl.when(t > 0)
                def _():
                    pltpu.make_async_copy(bufs[b], out_slab(j - NBUF), ssem[b]).wait()

                pltpu.async_copy(table_hbm.at[idx_v.at[j]], bufs[b], gsem[b])
            for b in range(NBUF):
                j = t * NBUF + b
                pltpu.make_async_copy(
                    table_hbm.at[idx_v.at[j]], bufs[b], gsem[b]
                ).wait()
                pltpu.async_copy(bufs[b], out_slab(j), ssem[b])
            return carry

        lax.fori_loop(0, NROUND, body, 0)
        for b in range(NBUF):
            pltpu.make_async_copy(bufs[b], out_slab((NROUND - 1) * NBUF + b), ssem[b]).wait()

    return gather_kernel


_gather = _make_gather()


def kernel(_, decoder_input_ids, probs):
    ids_t = decoder_input_ids[:, 1:].T.reshape(NITEM, G)
    idx = jnp.concatenate([ids_t, ids_t[: NW * IPW - NITEM]]).reshape(NW, IPW, G)
    out_t = _gather(idx, probs)
    return (out_t.transpose(1, 0, 2),)
